# baseline (speedup 1.0000x reference)
import jax
import jax.numpy as jnp
from jax.experimental import pallas as pl
from jax.experimental.pallas import tpu as pltpu

TOP_K_FRAC = 0.7
CB = 2048


def _make_lse_kernel(rows, vocab, nj, k):
    def _kern(x_ref, t_ref, out_ref, m_ref, s_ref, tv_ref):
        j = pl.program_id(0)

        @pl.when(j == 0)
        def _init():
            m_ref[...] = jnp.full((rows, 1), -jnp.inf, jnp.float32)
            s_ref[...] = jnp.zeros((rows, 1), jnp.float32)
            tv_ref[...] = jnp.zeros((rows, 1), jnp.float32)

        t = t_ref[...]

        def update(x, cols):
            bm = jnp.max(x, axis=1, keepdims=True)
            m_old = m_ref[...]
            m_new = jnp.maximum(m_old, bm)
            s_ref[...] = s_ref[...] * jnp.exp(m_old - m_new) + jnp.sum(
                jnp.exp(x - m_new), axis=1, keepdims=True)
            m_ref[...] = m_new
            tv_ref[...] += jnp.sum(jnp.where(cols == t, x, 0.0), axis=1,
                                   keepdims=True)

        @pl.when(j < nj - 1)
        def _full():
            cols = j * CB + jax.lax.broadcasted_iota(jnp.int32, (rows, CB), 1)
            update(x_ref[...], cols)

        @pl.when(j == nj - 1)
        def _tail():
            cols = j * CB + jax.lax.broadcasted_iota(jnp.int32, (rows, CB), 1)
            x = jnp.where(cols < vocab, x_ref[...], -jnp.inf)
            update(x, cols)

            loss = m_ref[...] + jnp.log(s_ref[...]) - tv_ref[...]
            u = jax.lax.bitcast_convert_type(loss, jnp.int32)

            def body(i, pfx):
                cand = pfx | jnp.left_shift(jnp.int32(1), 30 - i)
                cnt = jnp.sum((u >= cand).astype(jnp.int32))
                return jnp.where(cnt >= k, cand, pfx)

            thr = jax.lax.fori_loop(0, 31, body, jnp.int32(0))
            thr_f = jax.lax.bitcast_convert_type(thr, jnp.float32)
            gt = u > thr
            c_gt = jnp.sum(gt.astype(jnp.int32))
            s_top = jnp.sum(jnp.where(gt, loss, 0.0))
            out_ref[0, 0] = (s_top + (k - c_gt).astype(jnp.float32) * thr_f) / k

    return _kern


@jax.jit
def kernel(input, target):
    rows, vocab = input.shape
    nj = (vocab + CB - 1) // CB
    k = int(TOP_K_FRAC * rows)
    t = target.astype(jnp.int32).reshape(rows, 1)
    out = pl.pallas_call(
        _make_lse_kernel(rows, vocab, nj, k),
        grid=(nj,),
        in_specs=[
            pl.BlockSpec((rows, CB), lambda j: (0, j)),
            pl.BlockSpec((rows, 1), lambda j: (0, 0)),
        ],
        out_specs=pl.BlockSpec(memory_space=pltpu.SMEM),
        out_shape=jax.ShapeDtypeStruct((1, 1), jnp.float32),
        scratch_shapes=[
            pltpu.VMEM((rows, 1), jnp.float32),
            pltpu.VMEM((rows, 1), jnp.float32),
            pltpu.VMEM((rows, 1), jnp.float32),
        ],
        compiler_params=pltpu.CompilerParams(
            dimension_semantics=("arbitrary",),
        ),
    )(input, t)
    return out[0, 0]
